# baseline (speedup 1.0000x reference)
import functools

import jax
import jax.numpy as jnp
from jax import lax
from jax.experimental import pallas as pl
from jax.experimental.pallas import tpu as pltpu
from jax.experimental.pallas import tpu_sc as plsc

N, E, D, DE, H, Z = 10000, 320000, 128, 16, 1024, 64

LANES = 16
NC, NS = 2, 16
NW = NC * NS
EPW = E // NW
C = 40
NCH = EPW // C
ROWS_PER_SUB = 640
LAST_ROWS = N - (NS - 1) * ROWS_PER_SUB

EDGE_BLK = 6400
X_BLK = N // (E // EDGE_BLK)
N_BLK = 1000

_PREC = lax.Precision.DEFAULT


def _edge_lin_body(ea_ref, We_ref, be_ref, x_ref, o_ref, xo_ref):
    o_ref[...] = lax.dot_general(
        ea_ref[...], We_ref[...],
        dimension_numbers=(((0,), (0,)), ((), ())),
        preferred_element_type=jnp.float32,
        precision=_PREC) + be_ref[...]
    xo_ref[...] = x_ref[...]


_edge_lin = pl.pallas_call(
    _edge_lin_body,
    grid=(E // EDGE_BLK,),
    in_specs=[
        pl.BlockSpec((DE, EDGE_BLK), lambda i: (0, i)),
        pl.BlockSpec((DE, D), lambda i: (0, 0)),
        pl.BlockSpec((1, D), lambda i: (0, 0)),
        pl.BlockSpec((X_BLK, D), lambda i: (i, 0)),
    ],
    out_specs=[
        pl.BlockSpec((EDGE_BLK, D), lambda i: (i, 0)),
        pl.BlockSpec((X_BLK, D), lambda i: (i, 0)),
    ],
    out_shape=[
        jax.ShapeDtypeStruct((E, D), jnp.float32),
        jax.ShapeDtypeStruct((N, D), jnp.float32),
    ],
)


_mesh = plsc.VectorSubcoreMesh(core_axis_name="c", subcore_axis_name="s")


@functools.partial(
    pl.kernel,
    out_type=jax.ShapeDtypeStruct((NC, N, D), jnp.float32),
    mesh=_mesh,
    scratch_types=[
        pltpu.VMEM((C,), jnp.int32),
        pltpu.VMEM((C,), jnp.int32),
        pltpu.VMEM((C,), jnp.int32),
        pltpu.VMEM((C,), jnp.int32),
        pltpu.VMEM((C, D), jnp.float32),
        pltpu.VMEM((C, D), jnp.float32),
        pltpu.VMEM((C, D), jnp.float32),
        pltpu.VMEM((C, D), jnp.float32),
        pltpu.VMEM_SHARED((N, D), jnp.float32),
        pltpu.SemaphoreType.DMA,
        pltpu.SemaphoreType.DMA,
        pltpu.SemaphoreType.DMA,
        pltpu.SemaphoreType.DMA,
        pltpu.SemaphoreType.DMA,
        pltpu.SemaphoreType.DMA,
    ],
)
def _sc_aggregate(x_hbm, e_hbm, ei_hbm, out_hbm,
                  src0, src1, dst0, dst1, rows0, rows1, e0, e1, aggr_sh,
                  isem0, isem1, gsem0, gsem1, esem0, esem1):
    cid = lax.axis_index("c")
    sid = lax.axis_index("s")
    wid = sid * NC + cid
    ebase = wid * EPW
    zrow = sid * ROWS_PER_SUB
    nzrows = jnp.where(sid == NS - 1, LAST_ROWS, ROWS_PER_SUB)

    @pl.loop(0, C)
    def _(r):
        for j in range(0, D, LANES):
            rows0[r, pl.ds(j, LANES)] = jnp.zeros((LANES,), jnp.float32)

    @pl.loop(0, nzrows, step=C)
    def _(r0):
        pltpu.sync_copy(rows0, aggr_sh.at[pl.ds(zrow + r0, C)])

    def issue_idx(c, src_b, dst_b, isem_b):
        pltpu.async_copy(ei_hbm.at[0, wid, c], src_b, isem_b)
        pltpu.async_copy(ei_hbm.at[1, wid, c], dst_b, isem_b)

    def wait_idx(c, src_b, dst_b, isem_b):
        pltpu.make_async_copy(ei_hbm.at[0, wid, c], src_b, isem_b).wait()
        pltpu.make_async_copy(ei_hbm.at[1, wid, c], dst_b, isem_b).wait()

    def issue_data(c, src_b, rows_b, e_b, gsem_b, esem_b):
        pltpu.async_copy(x_hbm.at[src_b], rows_b, gsem_b)
        pltpu.async_copy(e_hbm.at[pl.ds(ebase + c * C, C)], e_b, esem_b)

    def process(c, src_b, dst_b, rows_b, e_b, gsem_b, esem_b):
        pltpu.make_async_copy(x_hbm.at[src_b], rows_b, gsem_b).wait()
        pltpu.make_async_copy(e_hbm.at[pl.ds(ebase + c * C, C)], e_b,
                              esem_b).wait()

        @pl.loop(0, C, unroll=4)
        def _(i):
            for j in range(0, D, LANES):
                s = pl.ds(j, LANES)
                rows_b[i, s] = jnp.maximum(rows_b[i, s] + e_b[i, s], 0.0)

        pltpu.sync_copy(rows_b, aggr_sh.at[dst_b], add=True)

    issue_idx(0, src0, dst0, isem0)
    issue_idx(1, src1, dst1, isem1)
    plsc.subcore_barrier()
    wait_idx(0, src0, dst0, isem0)
    issue_data(0, src0, rows0, e0, gsem0, esem0)

    @pl.loop(0, NCH, step=2)
    def _(k):
        wait_idx(k + 1, src1, dst1, isem1)
        issue_data(k + 1, src1, rows1, e1, gsem1, esem1)
        process(k, src0, dst0, rows0, e0, gsem0, esem0)

        @pl.when(k + 2 < NCH)
        def _():
            issue_idx(k + 2, src0, dst0, isem0)

        @pl.when(k + 2 < NCH)
        def _():
            wait_idx(k + 2, src0, dst0, isem0)
            issue_data(k + 2, src0, rows0, e0, gsem0, esem0)

        process(k + 1, src1, dst1, rows1, e1, gsem1, esem1)

        @pl.when(k + 3 < NCH)
        def _():
            issue_idx(k + 3, src1, dst1, isem1)

    plsc.subcore_barrier()

    @pl.when(sid < NS - 1)
    def _():
        pltpu.sync_copy(aggr_sh.at[pl.ds(zrow, ROWS_PER_SUB)],
                        out_hbm.at[cid, pl.ds(zrow, ROWS_PER_SUB)])

    @pl.when(sid == NS - 1)
    def _():
        pltpu.sync_copy(aggr_sh.at[pl.ds((NS - 1) * ROWS_PER_SUB, LAST_ROWS)],
                        out_hbm.at[cid, pl.ds((NS - 1) * ROWS_PER_SUB,
                                              LAST_ROWS)])


def _mlp_body(eps_ref, x_ref, a_ref, W1_ref, b1_ref, W2_ref, b2_ref, o_ref):
    h0 = (1.0 + eps_ref[0]) * x_ref[...] + (a_ref[0] + a_ref[1])
    h1 = jnp.maximum(jnp.dot(h0, W1_ref[...],
                             preferred_element_type=jnp.float32,
                             precision=_PREC) + b1_ref[...], 0.0)
    o_ref[...] = jnp.dot(h1, W2_ref[...],
                         preferred_element_type=jnp.float32,
                         precision=_PREC) + b2_ref[...]


_mlp = pl.pallas_call(
    _mlp_body,
    grid=(N // N_BLK,),
    in_specs=[
        pl.BlockSpec(memory_space=pltpu.SMEM),
        pl.BlockSpec((N_BLK, D), lambda i: (i, 0)),
        pl.BlockSpec((NC, N_BLK, D), lambda i: (0, i, 0)),
        pl.BlockSpec((D, H), lambda i: (0, 0)),
        pl.BlockSpec((1, H), lambda i: (0, 0)),
        pl.BlockSpec((H, Z), lambda i: (0, 0)),
        pl.BlockSpec((1, Z), lambda i: (0, 0)),
    ],
    out_specs=pl.BlockSpec((N_BLK, Z), lambda i: (i, 0)),
    out_shape=jax.ShapeDtypeStruct((N, Z), jnp.float32),
)


def kernel(x, edge_index, edge_attr, We, be, W1, b1, W2, b2, eps):
    e, x_sc = _edge_lin(edge_attr.T, We, be.reshape(1, D), x)
    ei = edge_index.reshape(2, NW, NCH, C)
    parts = _sc_aggregate(x_sc, e, ei)
    return _mlp(eps.reshape(1), x, parts, W1, b1.reshape(1, H),
                W2, b2.reshape(1, Z))


# speedup vs baseline: 1.3040x; 1.3040x over previous
import dataclasses
import functools

import jax
import jax.numpy as jnp
from jax import lax
from jax.experimental import pallas as pl
from jax.experimental.pallas import tpu as pltpu
from jax.experimental.pallas import tpu_sc as plsc

N, E, D, DE, H, Z = 10000, 320000, 128, 16, 1024, 64

LANES = 16
NC, NS = 2, 16
NW = NC * NS
EPW = E // NW
C = 40
NCH = EPW // C
ROWS_PER_SUB = 640
LAST_ROWS = N - (NS - 1) * ROWS_PER_SUB

EDGE_BLK = 6400
X_BLK = N // (E // EDGE_BLK)
N_BLK = 1000

_PREC = lax.Precision.DEFAULT


def _edge_lin_body(ea_ref, We_ref, be_ref, x_ref, o_ref, xo_ref):
    e = lax.dot_general(ea_ref[...], We_ref[...],
                        dimension_numbers=(((0,), (0,)), ((), ())),
                        preferred_element_type=jnp.float32,
                        precision=_PREC) + be_ref[...]
    o_ref[...] = e.astype(jnp.bfloat16)
    xo_ref[...] = x_ref[...]


_edge_lin = pl.pallas_call(
    _edge_lin_body,
    grid=(E // EDGE_BLK,),
    in_specs=[
        pl.BlockSpec((DE, EDGE_BLK), lambda i: (0, i)),
        pl.BlockSpec((DE, D), lambda i: (0, 0)),
        pl.BlockSpec((1, D), lambda i: (0, 0)),
        pl.BlockSpec((X_BLK, D), lambda i: (i, 0)),
    ],
    out_specs=[
        pl.BlockSpec((EDGE_BLK, D), lambda i: (i, 0)),
        pl.BlockSpec((X_BLK, D), lambda i: (i, 0)),
    ],
    out_shape=[
        jax.ShapeDtypeStruct((E, D), jnp.bfloat16),
        jax.ShapeDtypeStruct((N, D), jnp.float32),
    ],
)


_mesh = plsc.VectorSubcoreMesh(core_axis_name="c", subcore_axis_name="s")

_sc_params = pltpu.CompilerParams()
if "needs_layout_passes" in pltpu.CompilerParams.__dataclass_fields__:
    _sc_params = dataclasses.replace(_sc_params, needs_layout_passes=False)


@functools.partial(
    pl.kernel,
    out_type=jax.ShapeDtypeStruct((NC, N, D), jnp.float32),
    mesh=_mesh,
    compiler_params=_sc_params,
    scratch_types=[
        pltpu.VMEM((C,), jnp.int32),
        pltpu.VMEM((C,), jnp.int32),
        pltpu.VMEM((C,), jnp.int32),
        pltpu.VMEM((C,), jnp.int32),
        pltpu.VMEM((C, D), jnp.float32),
        pltpu.VMEM((C, D), jnp.float32),
        pltpu.VMEM((C, D), jnp.bfloat16),
        pltpu.VMEM((C, D), jnp.bfloat16),
        pltpu.VMEM_SHARED((N, D), jnp.float32),
        pltpu.SemaphoreType.DMA,
        pltpu.SemaphoreType.DMA,
        pltpu.SemaphoreType.DMA,
        pltpu.SemaphoreType.DMA,
        pltpu.SemaphoreType.DMA,
        pltpu.SemaphoreType.DMA,
    ],
)
def _sc_aggregate(x_hbm, e_hbm, ei_hbm, out_hbm,
                  src0, src1, dst0, dst1, rows0, rows1, e0, e1,
                  aggr_sh, isem0, isem1, gsem0, gsem1, esem0, esem1):
    cid = lax.axis_index("c")
    sid = lax.axis_index("s")
    wid = sid * NC + cid
    ebase = wid * EPW
    zrow = sid * ROWS_PER_SUB
    nzrows = jnp.where(sid == NS - 1, LAST_ROWS, ROWS_PER_SUB)

    @pl.loop(0, C)
    def _(r):
        for j in range(0, D, LANES):
            rows0[r, pl.ds(j, LANES)] = jnp.zeros((LANES,), jnp.float32)

    @pl.loop(0, nzrows, step=C)
    def _(r0):
        pltpu.sync_copy(rows0, aggr_sh.at[pl.ds(zrow + r0, C)])

    def issue_idx(c, src_b, dst_b, isem_b):
        pltpu.async_copy(ei_hbm.at[0, wid, c], src_b, isem_b)
        pltpu.async_copy(ei_hbm.at[1, wid, c], dst_b, isem_b)

    def wait_idx(c, src_b, dst_b, isem_b):
        pltpu.make_async_copy(ei_hbm.at[0, wid, c], src_b, isem_b).wait()
        pltpu.make_async_copy(ei_hbm.at[1, wid, c], dst_b, isem_b).wait()

    def issue_data(c, src_b, rows_b, e_b, gsem_b, esem_b):
        pltpu.async_copy(x_hbm.at[src_b], rows_b, gsem_b)
        pltpu.async_copy(e_hbm.at[pl.ds(ebase + c * C, C)], e_b, esem_b)

    def process(c, src_b, dst_b, rows_b, e_b, gsem_b, esem_b):
        pltpu.make_async_copy(x_hbm.at[src_b], rows_b, gsem_b).wait()
        pltpu.make_async_copy(e_hbm.at[pl.ds(ebase + c * C, C)], e_b,
                              esem_b).wait()

        zero16 = jnp.zeros((LANES,), jnp.float32)
        himask = jnp.full((LANES,), -65536, jnp.int32)

        e32 = e_b.bitcast(jnp.int32)
        ecols = D // 2

        @pl.loop(0, C // 8 * 4)
        def _(p):
            t = p // 4
            s = p - 4 * t
            r0 = 8 * t + s
            r1 = r0 + 4
            wbase = 512 * t + 128 * s
            for g in range(0, D, LANES):
                woff = wbase + g
                w = e32[woff // ecols, pl.ds(woff % ecols, LANES)]
                elo = plsc.bitcast(lax.shift_left(w, 16), jnp.float32)
                ehi = plsc.bitcast(lax.bitwise_and(w, himask), jnp.float32)
                sg = pl.ds(g, LANES)
                rows_b[r0, sg] = jnp.maximum(rows_b[r0, sg] + elo, zero16)
                rows_b[r1, sg] = jnp.maximum(rows_b[r1, sg] + ehi, zero16)

        pltpu.sync_copy(rows_b, aggr_sh.at[dst_b], add=True)

    issue_idx(0, src0, dst0, isem0)
    issue_idx(1, src1, dst1, isem1)
    plsc.subcore_barrier()
    wait_idx(0, src0, dst0, isem0)
    issue_data(0, src0, rows0, e0, gsem0, esem0)

    @pl.loop(0, NCH, step=2)
    def _(k):
        wait_idx(k + 1, src1, dst1, isem1)
        issue_data(k + 1, src1, rows1, e1, gsem1, esem1)
        process(k, src0, dst0, rows0, e0, gsem0, esem0)

        @pl.when(k + 2 < NCH)
        def _():
            issue_idx(k + 2, src0, dst0, isem0)

        @pl.when(k + 2 < NCH)
        def _():
            wait_idx(k + 2, src0, dst0, isem0)
            issue_data(k + 2, src0, rows0, e0, gsem0, esem0)

        process(k + 1, src1, dst1, rows1, e1, gsem1, esem1)

        @pl.when(k + 3 < NCH)
        def _():
            issue_idx(k + 3, src1, dst1, isem1)

    plsc.subcore_barrier()

    @pl.when(sid < NS - 1)
    def _():
        pltpu.sync_copy(aggr_sh.at[pl.ds(zrow, ROWS_PER_SUB)],
                        out_hbm.at[cid, pl.ds(zrow, ROWS_PER_SUB)])

    @pl.when(sid == NS - 1)
    def _():
        pltpu.sync_copy(aggr_sh.at[pl.ds((NS - 1) * ROWS_PER_SUB, LAST_ROWS)],
                        out_hbm.at[cid, pl.ds((NS - 1) * ROWS_PER_SUB,
                                              LAST_ROWS)])


def _mlp_body(eps_ref, x_ref, a_ref, W1_ref, b1_ref, W2_ref, b2_ref, o_ref):
    h0 = (1.0 + eps_ref[0]) * x_ref[...] + (a_ref[0] + a_ref[1])
    h1 = jnp.maximum(jnp.dot(h0, W1_ref[...],
                             preferred_element_type=jnp.float32,
                             precision=_PREC) + b1_ref[...], 0.0)
    o_ref[...] = jnp.dot(h1, W2_ref[...],
                         preferred_element_type=jnp.float32,
                         precision=_PREC) + b2_ref[...]


_mlp = pl.pallas_call(
    _mlp_body,
    grid=(N // N_BLK,),
    in_specs=[
        pl.BlockSpec(memory_space=pltpu.SMEM),
        pl.BlockSpec((N_BLK, D), lambda i: (i, 0)),
        pl.BlockSpec((NC, N_BLK, D), lambda i: (0, i, 0)),
        pl.BlockSpec((D, H), lambda i: (0, 0)),
        pl.BlockSpec((1, H), lambda i: (0, 0)),
        pl.BlockSpec((H, Z), lambda i: (0, 0)),
        pl.BlockSpec((1, Z), lambda i: (0, 0)),
    ],
    out_specs=pl.BlockSpec((N_BLK, Z), lambda i: (i, 0)),
    out_shape=jax.ShapeDtypeStruct((N, Z), jnp.float32),
)


def kernel(x, edge_index, edge_attr, We, be, W1, b1, W2, b2, eps):
    e, x_sc = _edge_lin(edge_attr.T, We, be.reshape(1, D), x)
    ei = edge_index.reshape(2, NW, NCH, C)
    parts = _sc_aggregate(x_sc, e, ei)
    return _mlp(eps.reshape(1), x, parts, W1, b1.reshape(1, H),
                W2, b2.reshape(1, Z))


# speedup vs baseline: 2.1339x; 1.6364x over previous
import functools

import jax
import jax.numpy as jnp
from jax import lax
from jax.experimental import pallas as pl
from jax.experimental.pallas import tpu as pltpu
from jax.experimental.pallas import tpu_sc as plsc

N, E, D, DE, H, Z = 10000, 320000, 128, 16, 1024, 64

LANES = 16
NC, NS = 2, 16
NW = NC * NS
EPW = E // NW
C = 40
NCH = EPW // C
ROWS_PER_SUB = 640
LAST_ROWS = N - (NS - 1) * ROWS_PER_SUB

EDGE_BLK = 6400
X_BLK = N // (E // EDGE_BLK)
N_BLK = 1000

_PREC = lax.Precision.DEFAULT


def _edge_lin_body(ea_ref, We_ref, be_ref, x_ref, o_ref, xo_ref):
    o_ref[...] = lax.dot_general(
        ea_ref[...], We_ref[...],
        dimension_numbers=(((0,), (0,)), ((), ())),
        preferred_element_type=jnp.float32,
        precision=_PREC) + be_ref[...]
    xo_ref[...] = x_ref[...]


_edge_lin = pl.pallas_call(
    _edge_lin_body,
    grid=(E // EDGE_BLK,),
    in_specs=[
        pl.BlockSpec((DE, EDGE_BLK), lambda i: (0, i)),
        pl.BlockSpec((DE, D), lambda i: (0, 0)),
        pl.BlockSpec((1, D), lambda i: (0, 0)),
        pl.BlockSpec((X_BLK, D), lambda i: (i, 0)),
    ],
    out_specs=[
        pl.BlockSpec((EDGE_BLK, D), lambda i: (i, 0)),
        pl.BlockSpec((X_BLK, D), lambda i: (i, 0)),
    ],
    out_shape=[
        jax.ShapeDtypeStruct((E, D), jnp.float32),
        jax.ShapeDtypeStruct((N, D), jnp.float32),
    ],
)


_mesh = plsc.VectorSubcoreMesh(core_axis_name="c", subcore_axis_name="s")


@functools.partial(
    pl.kernel,
    out_type=jax.ShapeDtypeStruct((NC, N, D), jnp.float32),
    mesh=_mesh,
    scratch_types=[
        [pltpu.VMEM((C,), jnp.int32) for _ in range(4)],
        [pltpu.VMEM((C,), jnp.int32) for _ in range(4)],
        pltpu.VMEM((C, D), jnp.float32),
        pltpu.VMEM((C, D), jnp.float32),
        pltpu.VMEM((C, D), jnp.float32),
        pltpu.VMEM((C, D), jnp.float32),
        pltpu.VMEM_SHARED((N, D), jnp.float32),
        [pltpu.SemaphoreType.DMA for _ in range(4)],
        pltpu.SemaphoreType.DMA,
        pltpu.SemaphoreType.DMA,
        pltpu.SemaphoreType.DMA,
        pltpu.SemaphoreType.DMA,
    ],
)
def _sc_aggregate(x_hbm, e_hbm, ei_hbm, out_hbm,
                  srcs, dsts, rows0, rows1, e0, e1, aggr_sh,
                  isems, gsem0, gsem1, esem0, esem1):
    cid = lax.axis_index("c")
    sid = lax.axis_index("s")
    wid = sid * NC + cid
    ebase = wid * EPW
    zrow = sid * ROWS_PER_SUB
    nzrows = jnp.where(sid == NS - 1, LAST_ROWS, ROWS_PER_SUB)

    @pl.loop(0, C)
    def _(r):
        for j in range(0, D, LANES):
            rows0[r, pl.ds(j, LANES)] = jnp.zeros((LANES,), jnp.float32)

    @pl.loop(0, nzrows, step=C)
    def _(r0):
        pltpu.sync_copy(rows0, aggr_sh.at[pl.ds(zrow + r0, C)])

    def issue_idx(c, slot):
        pltpu.async_copy(ei_hbm.at[0, wid, c], srcs[slot], isems[slot])
        pltpu.async_copy(ei_hbm.at[1, wid, c], dsts[slot], isems[slot])

    def wait_idx(c, slot):
        pltpu.make_async_copy(ei_hbm.at[0, wid, c], srcs[slot],
                              isems[slot]).wait()
        pltpu.make_async_copy(ei_hbm.at[1, wid, c], dsts[slot],
                              isems[slot]).wait()

    def issue_data(c, src_b, rows_b, e_b, gsem_b, esem_b):
        pltpu.async_copy(x_hbm.at[src_b], rows_b, gsem_b)
        pltpu.async_copy(e_hbm.at[pl.ds(ebase + c * C, C)], e_b, esem_b)

    def process(c, src_b, dst_b, rows_b, e_b, gsem_b, esem_b):
        pltpu.make_async_copy(x_hbm.at[src_b], rows_b, gsem_b).wait()
        pltpu.make_async_copy(e_hbm.at[pl.ds(ebase + c * C, C)], e_b,
                              esem_b).wait()

        @pl.loop(0, C)
        def _(i):
            for j in range(0, D, LANES):
                s = pl.ds(j, LANES)
                rows_b[i, s] = jnp.maximum(rows_b[i, s] + e_b[i, s], 0.0)

        pltpu.sync_copy(rows_b, aggr_sh.at[dst_b], add=True)

    issue_idx(0, 0)
    issue_idx(1, 1)
    issue_idx(2, 2)
    plsc.subcore_barrier()
    wait_idx(0, 0)
    issue_data(0, srcs[0], rows0, e0, gsem0, esem0)

    data = [(rows0, e0, gsem0, esem0), (rows1, e1, gsem1, esem1)]

    @pl.loop(0, NCH - 2, step=4)
    def _(k):
        issue_idx(k + 3, 3)
        for u in range(4):
            c = k + u
            rows_b, e_b, gsem_b, esem_b = data[u % 2]
            nrows_b, ne_b, ngsem_b, nesem_b = data[(u + 1) % 2]
            wait_idx(c + 1, (u + 1) % 4)
            issue_data(c + 1, srcs[(u + 1) % 4], nrows_b, ne_b,
                       ngsem_b, nesem_b)
            process(c, srcs[u % 4], dsts[u % 4], rows_b, e_b,
                    gsem_b, esem_b)
            if u < 3:
                nxt = c + 4
                if u == 2:
                    @pl.when(nxt < NCH)
                    def _():
                        issue_idx(nxt, (u + 4) % 4)
                else:
                    issue_idx(nxt, u % 4)

    wait_idx(NCH - 1, (NCH - 1) % 4)
    issue_data(NCH - 1, srcs[(NCH - 1) % 4], rows1, e1, gsem1, esem1)
    process(NCH - 2, srcs[(NCH - 2) % 4], dsts[(NCH - 2) % 4],
            rows0, e0, gsem0, esem0)
    process(NCH - 1, srcs[(NCH - 1) % 4], dsts[(NCH - 1) % 4],
            rows1, e1, gsem1, esem1)

    plsc.subcore_barrier()

    @pl.when(sid < NS - 1)
    def _():
        pltpu.sync_copy(aggr_sh.at[pl.ds(zrow, ROWS_PER_SUB)],
                        out_hbm.at[cid, pl.ds(zrow, ROWS_PER_SUB)])

    @pl.when(sid == NS - 1)
    def _():
        pltpu.sync_copy(aggr_sh.at[pl.ds((NS - 1) * ROWS_PER_SUB, LAST_ROWS)],
                        out_hbm.at[cid, pl.ds((NS - 1) * ROWS_PER_SUB,
                                              LAST_ROWS)])


def _mlp_body(eps_ref, x_ref, a_ref, W1_ref, b1_ref, W2_ref, b2_ref, o_ref):
    h0 = (1.0 + eps_ref[0]) * x_ref[...] + (a_ref[0] + a_ref[1])
    h1 = jnp.maximum(jnp.dot(h0, W1_ref[...],
                             preferred_element_type=jnp.float32,
                             precision=_PREC) + b1_ref[...], 0.0)
    o_ref[...] = jnp.dot(h1, W2_ref[...],
                         preferred_element_type=jnp.float32,
                         precision=_PREC) + b2_ref[...]


_mlp = pl.pallas_call(
    _mlp_body,
    grid=(N // N_BLK,),
    in_specs=[
        pl.BlockSpec(memory_space=pltpu.SMEM),
        pl.BlockSpec((N_BLK, D), lambda i: (i, 0)),
        pl.BlockSpec((NC, N_BLK, D), lambda i: (0, i, 0)),
        pl.BlockSpec((D, H), lambda i: (0, 0)),
        pl.BlockSpec((1, H), lambda i: (0, 0)),
        pl.BlockSpec((H, Z), lambda i: (0, 0)),
        pl.BlockSpec((1, Z), lambda i: (0, 0)),
    ],
    out_specs=pl.BlockSpec((N_BLK, Z), lambda i: (i, 0)),
    out_shape=jax.ShapeDtypeStruct((N, Z), jnp.float32),
)


def kernel(x, edge_index, edge_attr, We, be, W1, b1, W2, b2, eps):
    e, x_sc = _edge_lin(edge_attr.T, We, be.reshape(1, D), x)
    ei = edge_index.reshape(2, NW, NCH, C)
    parts = _sc_aggregate(x_sc, e, ei)
    return _mlp(eps.reshape(1), x, parts, W1, b1.reshape(1, H),
                W2, b2.reshape(1, Z))


# speedup vs baseline: 2.2246x; 1.0425x over previous
import functools

import jax
import jax.numpy as jnp
from jax import lax
from jax.experimental import pallas as pl
from jax.experimental.pallas import tpu as pltpu
from jax.experimental.pallas import tpu_sc as plsc

N, E, D, DE, H, Z = 10000, 320000, 128, 16, 1024, 64

LANES = 16
NC, NS = 2, 16
NW = NC * NS
EPW = E // NW
C = 40
NCH = EPW // C
ROWS_PER_SUB = 640
LAST_ROWS = N - (NS - 1) * ROWS_PER_SUB

EDGE_BLK = 12800
X_BLK = N // (E // EDGE_BLK)
N_BLK = 1000

_PREC = lax.Precision.DEFAULT


def _edge_lin_body(ea_ref, We_ref, be_ref, x_ref, o_ref, xo_ref):
    o_ref[...] = lax.dot_general(
        ea_ref[...], We_ref[...],
        dimension_numbers=(((0,), (0,)), ((), ())),
        preferred_element_type=jnp.float32,
        precision=_PREC) + be_ref[...]
    xo_ref[...] = x_ref[...]


_edge_lin = pl.pallas_call(
    _edge_lin_body,
    grid=(E // EDGE_BLK,),
    in_specs=[
        pl.BlockSpec((DE, EDGE_BLK), lambda i: (0, i)),
        pl.BlockSpec((DE, D), lambda i: (0, 0)),
        pl.BlockSpec((1, D), lambda i: (0, 0)),
        pl.BlockSpec((X_BLK, D), lambda i: (i, 0)),
    ],
    out_specs=[
        pl.BlockSpec((EDGE_BLK, D), lambda i: (i, 0)),
        pl.BlockSpec((X_BLK, D), lambda i: (i, 0)),
    ],
    out_shape=[
        jax.ShapeDtypeStruct((E, D), jnp.float32),
        jax.ShapeDtypeStruct((N, D), jnp.float32),
    ],
)


_mesh = plsc.VectorSubcoreMesh(core_axis_name="c", subcore_axis_name="s")


@functools.partial(
    pl.kernel,
    out_type=jax.ShapeDtypeStruct((NC, N, D), jnp.float32),
    mesh=_mesh,
    scratch_types=[
        [pltpu.VMEM((C,), jnp.int32) for _ in range(4)],
        [pltpu.VMEM((C,), jnp.int32) for _ in range(4)],
        pltpu.VMEM((C, D), jnp.float32),
        pltpu.VMEM((C, D), jnp.float32),
        pltpu.VMEM((C, D), jnp.float32),
        pltpu.VMEM((C, D), jnp.float32),
        pltpu.VMEM_SHARED((N, D), jnp.float32),
        [pltpu.SemaphoreType.DMA for _ in range(4)],
        pltpu.SemaphoreType.DMA,
        pltpu.SemaphoreType.DMA,
        pltpu.SemaphoreType.DMA,
        pltpu.SemaphoreType.DMA,
    ],
)
def _sc_aggregate(x_hbm, e_hbm, ei_hbm, out_hbm,
                  srcs, dsts, rows0, rows1, e0, e1, aggr_sh,
                  isems, gsem0, gsem1, esem0, esem1):
    cid = lax.axis_index("c")
    sid = lax.axis_index("s")
    wid = sid * NC + cid
    ebase = wid * EPW
    zrow = sid * ROWS_PER_SUB
    nzrows = jnp.where(sid == NS - 1, LAST_ROWS, ROWS_PER_SUB)

    @pl.loop(0, C)
    def _(r):
        for j in range(0, D, LANES):
            rows0[r, pl.ds(j, LANES)] = jnp.zeros((LANES,), jnp.float32)

    @pl.loop(0, nzrows, step=C)
    def _(r0):
        pltpu.sync_copy(rows0, aggr_sh.at[pl.ds(zrow + r0, C)])

    def issue_idx(c, slot):
        pltpu.async_copy(ei_hbm.at[0, wid, c], srcs[slot], isems[slot])
        pltpu.async_copy(ei_hbm.at[1, wid, c], dsts[slot], isems[slot])

    def wait_idx(c, slot):
        pltpu.make_async_copy(ei_hbm.at[0, wid, c], srcs[slot],
                              isems[slot]).wait()
        pltpu.make_async_copy(ei_hbm.at[1, wid, c], dsts[slot],
                              isems[slot]).wait()

    def issue_data(c, src_b, rows_b, e_b, gsem_b, esem_b):
        pltpu.async_copy(x_hbm.at[src_b], rows_b, gsem_b)
        pltpu.async_copy(e_hbm.at[pl.ds(ebase + c * C, C)], e_b, esem_b)

    def process(c, src_b, dst_b, rows_b, e_b, gsem_b, esem_b):
        pltpu.make_async_copy(x_hbm.at[src_b], rows_b, gsem_b).wait()
        pltpu.make_async_copy(e_hbm.at[pl.ds(ebase + c * C, C)], e_b,
                              esem_b).wait()

        @pl.loop(0, C)
        def _(i):
            for j in range(0, D, LANES):
                s = pl.ds(j, LANES)
                rows_b[i, s] = jnp.maximum(rows_b[i, s] + e_b[i, s], 0.0)

        pltpu.sync_copy(rows_b, aggr_sh.at[dst_b], add=True)

    issue_idx(0, 0)
    issue_idx(1, 1)
    issue_idx(2, 2)
    plsc.subcore_barrier()
    wait_idx(0, 0)
    issue_data(0, srcs[0], rows0, e0, gsem0, esem0)

    data = [(rows0, e0, gsem0, esem0), (rows1, e1, gsem1, esem1)]

    @pl.loop(0, NCH - 2, step=4)
    def _(k):
        issue_idx(k + 3, 3)
        for u in range(4):
            c = k + u
            rows_b, e_b, gsem_b, esem_b = data[u % 2]
            nrows_b, ne_b, ngsem_b, nesem_b = data[(u + 1) % 2]
            wait_idx(c + 1, (u + 1) % 4)
            issue_data(c + 1, srcs[(u + 1) % 4], nrows_b, ne_b,
                       ngsem_b, nesem_b)
            process(c, srcs[u % 4], dsts[u % 4], rows_b, e_b,
                    gsem_b, esem_b)
            if u < 3:
                nxt = c + 4
                if u == 2:
                    @pl.when(nxt < NCH)
                    def _():
                        issue_idx(nxt, (u + 4) % 4)
                else:
                    issue_idx(nxt, u % 4)

    wait_idx(NCH - 1, (NCH - 1) % 4)
    issue_data(NCH - 1, srcs[(NCH - 1) % 4], rows1, e1, gsem1, esem1)
    process(NCH - 2, srcs[(NCH - 2) % 4], dsts[(NCH - 2) % 4],
            rows0, e0, gsem0, esem0)
    process(NCH - 1, srcs[(NCH - 1) % 4], dsts[(NCH - 1) % 4],
            rows1, e1, gsem1, esem1)

    plsc.subcore_barrier()

    @pl.when(sid < NS - 1)
    def _():
        pltpu.sync_copy(aggr_sh.at[pl.ds(zrow, ROWS_PER_SUB)],
                        out_hbm.at[cid, pl.ds(zrow, ROWS_PER_SUB)])

    @pl.when(sid == NS - 1)
    def _():
        pltpu.sync_copy(aggr_sh.at[pl.ds((NS - 1) * ROWS_PER_SUB, LAST_ROWS)],
                        out_hbm.at[cid, pl.ds((NS - 1) * ROWS_PER_SUB,
                                              LAST_ROWS)])


def _mlp_body(eps_ref, x_ref, a_ref, W1_ref, b1_ref, W2_ref, b2_ref, o_ref):
    h0 = (1.0 + eps_ref[0]) * x_ref[...] + (a_ref[0] + a_ref[1])
    h1 = jnp.maximum(jnp.dot(h0, W1_ref[...],
                             preferred_element_type=jnp.float32,
                             precision=_PREC) + b1_ref[...], 0.0)
    o_ref[...] = jnp.dot(h1, W2_ref[...],
                         preferred_element_type=jnp.float32,
                         precision=_PREC) + b2_ref[...]


_mlp = pl.pallas_call(
    _mlp_body,
    grid=(N // N_BLK,),
    in_specs=[
        pl.BlockSpec(memory_space=pltpu.SMEM),
        pl.BlockSpec((N_BLK, D), lambda i: (i, 0)),
        pl.BlockSpec((NC, N_BLK, D), lambda i: (0, i, 0)),
        pl.BlockSpec((D, H), lambda i: (0, 0)),
        pl.BlockSpec((1, H), lambda i: (0, 0)),
        pl.BlockSpec((H, Z), lambda i: (0, 0)),
        pl.BlockSpec((1, Z), lambda i: (0, 0)),
    ],
    out_specs=pl.BlockSpec((N_BLK, Z), lambda i: (i, 0)),
    out_shape=jax.ShapeDtypeStruct((N, Z), jnp.float32),
)


def kernel(x, edge_index, edge_attr, We, be, W1, b1, W2, b2, eps):
    e, x_sc = _edge_lin(edge_attr.T, We, be.reshape(1, D), x)
    ei = edge_index.reshape(2, NW, NCH, C)
    parts = _sc_aggregate(x_sc, e, ei)
    return _mlp(eps.reshape(1), x, parts, W1, b1.reshape(1, H),
                W2, b2.reshape(1, Z))
